# baseline (speedup 1.0000x reference)
import functools

import jax
import jax.numpy as jnp
from jax import lax
from jax.experimental import pallas as pl
from jax.experimental.pallas import tpu as pltpu
from jax.experimental.pallas import tpu_sc as plsc

_N = 10000
_D = 128
_F = 4
_G = 128
_C = 10
_BM = 400
_NBLK = _N // _BM

_NW = 32
_NP = 10240
_CHUNK = _NP // _NW
_NVEC = _CHUNK // 16
_GP = 136
_BUF = _F * _GP


def _stream(filtre_ref, x_ref, w_ref, b_ref, h_ref, y_scr):
    i = pl.program_id(0)

    @pl.when(i == 0)
    def _init():
        y_scr[...] = jnp.dot(x_ref[...], w_ref[...],
                             preferred_element_type=jnp.float32
                             ).astype(jnp.bfloat16)

    fblk = filtre_ref[...].astype(jnp.bfloat16)
    h = jnp.dot(fblk, y_scr[...],
                preferred_element_type=jnp.float32)
    h = jnp.maximum(h + b_ref[...], 0.0)
    h_ref[...] = h


def _lane_shuffle(x, src):
    dnums = lax.GatherDimensionNumbers(
        offset_dims=(), collapsed_slice_dims=(0,), start_index_map=(0,))
    return lax.gather(x, src[:, None], dnums, slice_sizes=(1,),
                      mode=lax.GatherScatterMode.PROMISE_IN_BOUNDS)


def _sc_pool(h_hbm, ind_hbm, out_hbm, h_v, ind_v, buf_v):
    wid = lax.axis_index("s") * 2 + lax.axis_index("c")
    base = wid * _CHUNK
    pltpu.sync_copy(ind_hbm.at[pl.ds(base, _CHUNK)], ind_v)
    pltpu.sync_copy(h_hbm.at[pl.ds(base * _F, _CHUNK * _F)], h_v)

    zero16 = jnp.zeros((16,), jnp.float32)
    for j in range(_BUF // 16):
        buf_v[pl.ds(j * 16, 16)] = zero16

    lane = lax.iota(jnp.int32, 16)
    nxt = jnp.minimum(lane + 1, 15)
    shift_src = [jnp.maximum(lane - st, 0) for st in (1, 2, 4, 8)]

    for j in range(_NVEC):
        idx = ind_v[pl.ds(j * 16, 16)]
        idx_sh = [_lane_shuffle(idx, s) for s in shift_src]
        idx_n = _lane_shuffle(idx, nxt)
        last = (idx != idx_n) | (lane == 15)
        rows = (lane + j * 16) * _F
        for f in range(_F):
            v = plsc.load_gather(h_v, [rows + f])
            for s, i_s in zip(shift_src, idx_sh):
                v_s = _lane_shuffle(v, s)
                v = jnp.where(idx == i_s, jnp.maximum(v, v_s), v)
            flat = idx + f * _GP
            old = plsc.load_gather(buf_v, [flat], mask=last)
            merged = jnp.maximum(v, old)
            plsc.store_scatter(buf_v, [flat], merged, mask=last)

    pltpu.sync_copy(buf_v, out_hbm.at[wid])


def _finalize(part_ref, wc_ref, bc_ref, out_ref):
    red = jnp.max(part_ref[...], axis=0, keepdims=True)
    pooled = jnp.concatenate(
        [red[:, f * _GP:f * _GP + _G] for f in range(_F)], axis=0)
    logits = jax.lax.dot_general(
        pooled, wc_ref[...], (((0,), (0,)), ((), ())),
        preferred_element_type=jnp.float32) + bc_ref[...]
    m = jnp.max(logits, axis=1, keepdims=True)
    e = jnp.exp(logits - m)
    out_ref[...] = e / jnp.sum(e, axis=1, keepdims=True)


def kernel(filtre, X, node_indicator, W, b, Wc, bc):
    b2d = b.reshape(1, _F)
    bc2d = bc.reshape(1, _C)

    h = pl.pallas_call(
        _stream,
        grid=(_NBLK,),
        in_specs=[
            pl.BlockSpec((_BM, _N), lambda i: (i, 0)),
            pl.BlockSpec((_N, _D), lambda i: (0, 0)),
            pl.BlockSpec((_D, _F), lambda i: (0, 0)),
            pl.BlockSpec((1, _F), lambda i: (0, 0)),
        ],
        out_specs=pl.BlockSpec((_BM, _F), lambda i: (i, 0)),
        out_shape=jax.ShapeDtypeStruct((_N, _F), jnp.float32),
        scratch_shapes=[pltpu.VMEM((_N, _F), jnp.bfloat16)],
        compiler_params=pltpu.CompilerParams(
            dimension_semantics=("arbitrary",),
            vmem_limit_bytes=100 * 1024 * 1024,
        ),
    )(filtre, X, W, b2d)

    h_pad = jnp.pad(h.reshape(-1), (0, (_NP - _N) * _F))
    ind_pad = jnp.concatenate([
        node_indicator.astype(jnp.int32),
        jnp.full((_NP - _N,), _G, jnp.int32),
    ])

    sc_pool = functools.partial(
        pl.kernel,
        mesh=plsc.VectorSubcoreMesh(core_axis_name="c", subcore_axis_name="s"),
        out_type=jax.ShapeDtypeStruct((_NW, _BUF), jnp.float32),
        scratch_types=[
            pltpu.VMEM((_CHUNK * _F,), jnp.float32),
            pltpu.VMEM((_CHUNK,), jnp.int32),
            pltpu.VMEM((_BUF,), jnp.float32),
        ],
        compiler_params=pltpu.CompilerParams(needs_layout_passes=False),
    )(_sc_pool)
    part = sc_pool(h_pad, ind_pad)

    return pl.pallas_call(
        _finalize,
        in_specs=[
            pl.BlockSpec((_NW, _BUF), lambda: (0, 0)),
            pl.BlockSpec((_F, _C), lambda: (0, 0)),
            pl.BlockSpec((1, _C), lambda: (0, 0)),
        ],
        out_specs=pl.BlockSpec((_G, _C), lambda: (0, 0)),
        out_shape=jax.ShapeDtypeStruct((_G, _C), jnp.float32),
    )(part, Wc, bc2d)


# speedup vs baseline: 1.0003x; 1.0003x over previous
import functools

import jax
import jax.numpy as jnp
from jax import lax
from jax.experimental import pallas as pl
from jax.experimental.pallas import tpu as pltpu
from jax.experimental.pallas import tpu_sc as plsc

_N = 10000
_D = 128
_F = 4
_G = 128
_C = 10
_BM = 400
_NBLK = _N // _BM

_NW = 32
_NP = 10240
_CHUNK = _NP // _NW
_NVEC = _CHUNK // 16
_GP = 136
_BUF = _F * _GP


def _stream(filtre_ref, x_ref, w_ref, b_ref, h_ref, y_scr):
    i = pl.program_id(0)

    @pl.when(i == 0)
    def _init():
        y_scr[...] = jnp.dot(x_ref[...], w_ref[...],
                             preferred_element_type=jnp.float32
                             ).astype(jnp.bfloat16)

    fblk = filtre_ref[...].astype(jnp.bfloat16)
    h = jnp.dot(fblk, y_scr[...],
                preferred_element_type=jnp.float32)
    h = jnp.maximum(h + b_ref[...], 0.0)
    h_ref[...] = h


def _lane_shuffle(x, src):
    dnums = lax.GatherDimensionNumbers(
        offset_dims=(), collapsed_slice_dims=(0,), start_index_map=(0,))
    return lax.gather(x, src[:, None], dnums, slice_sizes=(1,),
                      mode=lax.GatherScatterMode.PROMISE_IN_BOUNDS)


_TAILW = _NW - 1
_TAILROWS = _N - _TAILW * _CHUNK
_TAILVEC = _TAILROWS // 16


def _sc_pool(h_hbm, ind_hbm, out_hbm, h_v, ind_v, buf_v):
    wid = lax.axis_index("s") * 2 + lax.axis_index("c")
    base = wid * _CHUNK

    zero16 = jnp.zeros((16,), jnp.float32)
    for j in range(_BUF // 16):
        buf_v[pl.ds(j * 16, 16)] = zero16

    lane = lax.iota(jnp.int32, 16)
    nxt = jnp.minimum(lane + 1, 15)
    shift_src = [jnp.maximum(lane - st, 0) for st in (1, 2, 4, 8)]

    def run(nvec):
        for j in range(nvec):
            idx = ind_v[pl.ds(j * 16, 16)]
            idx_sh = [_lane_shuffle(idx, s) for s in shift_src]
            idx_n = _lane_shuffle(idx, nxt)
            last = (idx != idx_n) | (lane == 15)
            rows = (lane + j * 16) * _F
            for f in range(_F):
                v = plsc.load_gather(h_v, [rows + f])
                for s, i_s in zip(shift_src, idx_sh):
                    v_s = _lane_shuffle(v, s)
                    v = jnp.where(idx == i_s, jnp.maximum(v, v_s), v)
                flat = idx + f * _GP
                old = plsc.load_gather(buf_v, [flat], mask=last)
                merged = jnp.maximum(v, old)
                plsc.store_scatter(buf_v, [flat], merged, mask=last)

    @pl.when(wid < _TAILW)
    def _full():
        pltpu.sync_copy(ind_hbm.at[pl.ds(base, _CHUNK)], ind_v)
        pltpu.sync_copy(h_hbm.at[pl.ds(base * _F, _CHUNK * _F)], h_v)
        run(_NVEC)

    @pl.when(wid == _TAILW)
    def _tail():
        pltpu.sync_copy(ind_hbm.at[pl.ds(_TAILW * _CHUNK, _TAILROWS)],
                        ind_v.at[pl.ds(0, _TAILROWS)])
        pltpu.sync_copy(h_hbm.at[pl.ds(_TAILW * _CHUNK * _F, _TAILROWS * _F)],
                        h_v.at[pl.ds(0, _TAILROWS * _F)])
        run(_TAILVEC)

    pltpu.sync_copy(buf_v, out_hbm.at[wid])


def _finalize(part_ref, wc_ref, bc_ref, out_ref):
    red = jnp.max(part_ref[...], axis=0, keepdims=True)
    pooled = jnp.concatenate(
        [red[:, f * _GP:f * _GP + _G] for f in range(_F)], axis=0)
    logits = jax.lax.dot_general(
        pooled, wc_ref[...], (((0,), (0,)), ((), ())),
        preferred_element_type=jnp.float32) + bc_ref[...]
    m = jnp.max(logits, axis=1, keepdims=True)
    e = jnp.exp(logits - m)
    out_ref[...] = e / jnp.sum(e, axis=1, keepdims=True)


def kernel(filtre, X, node_indicator, W, b, Wc, bc):
    b2d = b.reshape(1, _F)
    bc2d = bc.reshape(1, _C)

    h = pl.pallas_call(
        _stream,
        grid=(_NBLK,),
        in_specs=[
            pl.BlockSpec((_BM, _N), lambda i: (i, 0)),
            pl.BlockSpec((_N, _D), lambda i: (0, 0)),
            pl.BlockSpec((_D, _F), lambda i: (0, 0)),
            pl.BlockSpec((1, _F), lambda i: (0, 0)),
        ],
        out_specs=pl.BlockSpec((_BM, _F), lambda i: (i, 0)),
        out_shape=jax.ShapeDtypeStruct((_NP, _F), jnp.float32),
        scratch_shapes=[pltpu.VMEM((_N, _F), jnp.bfloat16)],
        compiler_params=pltpu.CompilerParams(
            dimension_semantics=("arbitrary",),
            vmem_limit_bytes=100 * 1024 * 1024,
        ),
    )(filtre, X, W, b2d)

    h_flat = h.reshape(_NP * _F)
    ind32 = node_indicator.astype(jnp.int32)

    sc_pool = functools.partial(
        pl.kernel,
        mesh=plsc.VectorSubcoreMesh(core_axis_name="c", subcore_axis_name="s"),
        out_type=jax.ShapeDtypeStruct((_NW, _BUF), jnp.float32),
        scratch_types=[
            pltpu.VMEM((_CHUNK * _F,), jnp.float32),
            pltpu.VMEM((_CHUNK,), jnp.int32),
            pltpu.VMEM((_BUF,), jnp.float32),
        ],
        compiler_params=pltpu.CompilerParams(needs_layout_passes=False),
    )(_sc_pool)
    part = sc_pool(h_flat, ind32)

    return pl.pallas_call(
        _finalize,
        in_specs=[
            pl.BlockSpec((_NW, _BUF), lambda: (0, 0)),
            pl.BlockSpec((_F, _C), lambda: (0, 0)),
            pl.BlockSpec((1, _C), lambda: (0, 0)),
        ],
        out_specs=pl.BlockSpec((_G, _C), lambda: (0, 0)),
        out_shape=jax.ShapeDtypeStruct((_G, _C), jnp.float32),
    )(part, Wc, bc2d)


# speedup vs baseline: 1.0239x; 1.0236x over previous
import functools

import jax
import jax.numpy as jnp
from jax import lax
from jax.experimental import pallas as pl
from jax.experimental.pallas import tpu as pltpu
from jax.experimental.pallas import tpu_sc as plsc

_N = 10000
_D = 128
_F = 4
_G = 128
_C = 10
_BM = 256
_NBLK = 40

_NW = 32
_NP = 10240
_CHUNK = _NP // _NW
_NVEC = _CHUNK // 16
_GP = 136
_BUF = _F * _GP


def _stream(filtre_ref, x_ref, w_ref, b_ref, h_ref, y_scr):
    i = pl.program_id(0)

    @pl.when(i == 0)
    def _init():
        y_scr[...] = jnp.dot(x_ref[...], w_ref[...],
                             preferred_element_type=jnp.float32
                             ).astype(jnp.bfloat16)

    fblk = filtre_ref[...].astype(jnp.bfloat16)
    h = jnp.dot(fblk, y_scr[...],
                preferred_element_type=jnp.float32)
    h = jnp.maximum(h + b_ref[...], 0.0)
    h_ref[...] = h.T


def _lane_shuffle(x, src):
    dnums = lax.GatherDimensionNumbers(
        offset_dims=(), collapsed_slice_dims=(0,), start_index_map=(0,))
    return lax.gather(x, src[:, None], dnums, slice_sizes=(1,),
                      mode=lax.GatherScatterMode.PROMISE_IN_BOUNDS)


_TAILW = _NW - 1
_TAILROWS = _N - _TAILW * _CHUNK
_TAILVEC = _TAILROWS // 16


def _sc_pool(h_hbm, ind_hbm, out_hbm, h_v, ind_v, buf_v):
    wid = lax.axis_index("s") * 2 + lax.axis_index("c")
    base = wid * _CHUNK

    zero16 = jnp.zeros((16,), jnp.float32)
    for j in range(_BUF // 16):
        buf_v[pl.ds(j * 16, 16)] = zero16

    lane = lax.iota(jnp.int32, 16)
    nxt = jnp.minimum(lane + 1, 15)
    shift_src = [jnp.maximum(lane - st, 0) for st in (1, 2, 4, 8)]

    def run(nvec):
        for j in range(nvec):
            idx = ind_v[pl.ds(j * 16, 16)]
            idx_sh = [_lane_shuffle(idx, s) for s in shift_src]
            idx_n = _lane_shuffle(idx, nxt)
            last = (idx != idx_n) | (lane == 15)
            for f in range(_F):
                v = h_v[pl.ds(f * _CHUNK + j * 16, 16)]
                for s, i_s in zip(shift_src, idx_sh):
                    v_s = _lane_shuffle(v, s)
                    v = jnp.where(idx == i_s, jnp.maximum(v, v_s), v)
                flat = idx + f * _GP
                old = plsc.load_gather(buf_v, [flat], mask=last)
                merged = jnp.maximum(v, old)
                plsc.store_scatter(buf_v, [flat], merged, mask=last)

    @pl.when(wid < _TAILW)
    def _full():
        pltpu.sync_copy(ind_hbm.at[pl.ds(base, _CHUNK)], ind_v)
        for f in range(_F):
            pltpu.sync_copy(h_hbm.at[pl.ds(f * _NP + base, _CHUNK)],
                            h_v.at[pl.ds(f * _CHUNK, _CHUNK)])
        run(_NVEC)

    @pl.when(wid == _TAILW)
    def _tail():
        pltpu.sync_copy(ind_hbm.at[pl.ds(_TAILW * _CHUNK, _TAILROWS)],
                        ind_v.at[pl.ds(0, _TAILROWS)])
        for f in range(_F):
            pltpu.sync_copy(h_hbm.at[pl.ds(f * _NP + _TAILW * _CHUNK, _TAILROWS)],
                            h_v.at[pl.ds(f * _CHUNK, _TAILROWS)])
        run(_TAILVEC)

    pltpu.sync_copy(buf_v, out_hbm.at[wid])


def _finalize(part_ref, wc_ref, bc_ref, out_ref):
    red = jnp.max(part_ref[...], axis=0, keepdims=True)
    pooled = jnp.concatenate(
        [red[:, f * _GP:f * _GP + _G] for f in range(_F)], axis=0)
    logits = jax.lax.dot_general(
        pooled, wc_ref[...], (((0,), (0,)), ((), ())),
        preferred_element_type=jnp.float32) + bc_ref[...]
    m = jnp.max(logits, axis=1, keepdims=True)
    e = jnp.exp(logits - m)
    out_ref[...] = e / jnp.sum(e, axis=1, keepdims=True)


def kernel(filtre, X, node_indicator, W, b, Wc, bc):
    b2d = b.reshape(1, _F)
    bc2d = bc.reshape(1, _C)

    h = pl.pallas_call(
        _stream,
        grid=(_NBLK,),
        in_specs=[
            pl.BlockSpec((_BM, _N), lambda i: (i, 0)),
            pl.BlockSpec((_N, _D), lambda i: (0, 0)),
            pl.BlockSpec((_D, _F), lambda i: (0, 0)),
            pl.BlockSpec((1, _F), lambda i: (0, 0)),
        ],
        out_specs=pl.BlockSpec((_F, _BM), lambda i: (0, i)),
        out_shape=jax.ShapeDtypeStruct((_F, _NP), jnp.float32),
        scratch_shapes=[pltpu.VMEM((_N, _F), jnp.bfloat16)],
        compiler_params=pltpu.CompilerParams(
            dimension_semantics=("arbitrary",),
            vmem_limit_bytes=100 * 1024 * 1024,
        ),
    )(filtre, X, W, b2d)

    h_flat = h.reshape(_F * _NP)
    ind32 = node_indicator.astype(jnp.int32)

    sc_pool = functools.partial(
        pl.kernel,
        mesh=plsc.VectorSubcoreMesh(core_axis_name="c", subcore_axis_name="s"),
        out_type=jax.ShapeDtypeStruct((_NW, _BUF), jnp.float32),
        scratch_types=[
            pltpu.VMEM((_CHUNK * _F,), jnp.float32),
            pltpu.VMEM((_CHUNK,), jnp.int32),
            pltpu.VMEM((_BUF,), jnp.float32),
        ],
        compiler_params=pltpu.CompilerParams(needs_layout_passes=False),
    )(_sc_pool)
    part = sc_pool(h_flat, ind32)

    return pl.pallas_call(
        _finalize,
        in_specs=[
            pl.BlockSpec((_NW, _BUF), lambda: (0, 0)),
            pl.BlockSpec((_F, _C), lambda: (0, 0)),
            pl.BlockSpec((1, _C), lambda: (0, 0)),
        ],
        out_specs=pl.BlockSpec((_G, _C), lambda: (0, 0)),
        out_shape=jax.ShapeDtypeStruct((_G, _C), jnp.float32),
    )(part, Wc, bc2d)
